# baseline (speedup 1.0000x reference)
import functools

import jax
import jax.numpy as jnp
from jax import lax
from jax.experimental import pallas as pl
from jax.experimental.pallas import tpu as pltpu
from jax.experimental.pallas import tpu_sc as plsc

LANES = 16
NW = 32
CN = 4


def _tc_tables(x_t, a_t, b_t, bias):
    np_, c = x_t.shape
    out = a_t.shape[1]

    def body(x_ref, a_ref, bt_ref, bias_ref, u_ref, v_ref):
        xb = x_ref[...]
        u_ref[...] = (
            jnp.dot(xb, a_ref[...], preferred_element_type=jnp.float32)
            + bias_ref[...]
        ).astype(jnp.bfloat16)
        v_ref[...] = jnp.dot(
            xb, bt_ref[...], preferred_element_type=jnp.float32
        ).astype(jnp.bfloat16)

    return pl.pallas_call(
        body,
        out_shape=[
            jax.ShapeDtypeStruct((np_, out), jnp.bfloat16),
            jax.ShapeDtypeStruct((np_, out), jnp.bfloat16),
        ],
    )(x_t, a_t, b_t, bias)


def _sc_aggregate(t, idx_i, idx_j, n_pad, out_dim, k):
    pw = n_pad // NW
    n_chunks = pw // CN
    ce = CN * k
    wrow = out_dim // 2
    groups = wrow // LANES

    mesh = plsc.VectorSubcoreMesh(core_axis_name="c", subcore_axis_name="s")
    D = 3

    @functools.partial(
        pl.kernel,
        mesh=mesh,
        out_type=jax.ShapeDtypeStruct((n_pad, wrow), jnp.int32),
        scratch_types=[
            [pltpu.VMEM((ce,), jnp.int32)] * D,
            [pltpu.VMEM((ce,), jnp.int32)] * D,
            [pltpu.VMEM((ce, out_dim), jnp.int32)] * D,
            [pltpu.VMEM((ce, out_dim), jnp.int32)] * D,
            [pltpu.VMEM((CN, wrow), jnp.int32)] * D,
            [pltpu.SemaphoreType.DMA] * D,
            [pltpu.SemaphoreType.DMA] * D,
            [pltpu.SemaphoreType.DMA] * D,
            [pltpu.SemaphoreType.DMA] * D,
            [pltpu.SemaphoreType.DMA] * D,
        ],
    )
    def sc_kernel(t_hbm, ii_hbm, jj_hbm, out_hbm,
                  ii_v, jj_v, u_v, v_v, o_v,
                  sem_ii, sem_jj, sem_u, sem_v, sem_o):
        wid = lax.axis_index("s") * 2 + lax.axis_index("c")
        base = wid * pw

        def idx_start(ci, buf):
            es = (base + ci * CN) * k
            pltpu.make_async_copy(ii_hbm.at[pl.ds(es, ce)], ii_v[buf], sem_ii[buf]).start()
            pltpu.make_async_copy(jj_hbm.at[pl.ds(es, ce)], jj_v[buf], sem_jj[buf]).start()

        def idx_wait(buf):
            pltpu.make_async_copy(ii_hbm.at[pl.ds(0, ce)], ii_v[buf], sem_ii[buf]).wait()
            pltpu.make_async_copy(jj_hbm.at[pl.ds(0, ce)], jj_v[buf], sem_jj[buf]).wait()

        def gather_start(buf):
            pltpu.make_async_copy(t_hbm.at[ii_v[buf]], u_v[buf], sem_u[buf]).start()
            pltpu.make_async_copy(t_hbm.at[jj_v[buf]], v_v[buf], sem_v[buf]).start()

        def gather_wait(buf):
            pltpu.make_async_copy(t_hbm.at[ii_v[buf]], u_v[buf], sem_u[buf]).wait()
            pltpu.make_async_copy(t_hbm.at[jj_v[buf]], v_v[buf], sem_v[buf]).wait()

        for d in range(D):
            idx_start(d, d)
        for d in range(D - 1):
            idx_wait(d)
            gather_start(d)

        def iteration(ci, b):
            @pl.when(ci + D - 1 < n_chunks)
            def _():
                idx_wait((b + D - 1) % D)
                gather_start((b + D - 1) % D)

            gather_wait(b)

            @pl.when(ci + D < n_chunks)
            def _():
                idx_start(ci + D, b)

            @pl.when(ci >= D)
            def _():
                pltpu.make_async_copy(
                    o_v[b], out_hbm.at[pl.ds(base, CN)], sem_o[b]).wait()

            hi_mask = jnp.int32(-65536)
            for n in range(CN):
                for g in range(groups):
                    sl = pl.ds(g * LANES, LANES)
                    se = []
                    so = []
                    slv = pl.ds(wrow + g * LANES, LANES)
                    for kk in range(k):
                        uw = u_v[b][n * k + kk, sl]
                        vw = v_v[b][n * k + kk, slv]
                        ue = lax.bitcast_convert_type(uw << 16, jnp.float32)
                        uo = lax.bitcast_convert_type(uw & hi_mask, jnp.float32)
                        ve = lax.bitcast_convert_type(vw << 16, jnp.float32)
                        vo = lax.bitcast_convert_type(vw & hi_mask, jnp.float32)
                        se.append(ue + ve)
                        so.append(uo + vo)
                    for s in (se, so):
                        while len(s) > 1:
                            s2 = [jnp.maximum(s[2 * i], s[2 * i + 1])
                                  for i in range(len(s) // 2)]
                            s2 += s[len(s) & ~1:]
                            s[:] = s2
                    xe = lax.bitcast_convert_type(jnp.maximum(se[0], 0.0), jnp.int32)
                    xo = lax.bitcast_convert_type(jnp.maximum(so[0], 0.0), jnp.int32)
                    re = (xe + 32767 + ((xe >> 16) & 1)) >> 16
                    ro = (xo + 32767 + ((xo >> 16) & 1)) & hi_mask
                    o_v[b][n, sl] = re | ro

            ns = base + ci * CN
            pltpu.make_async_copy(o_v[b], out_hbm.at[pl.ds(ns, CN)], sem_o[b]).start()

        def body(p, carry):
            for j in range(D):
                iteration(p * D + j, j)
            return carry

        lax.fori_loop(0, n_chunks // D, body, 0)
        for ci in range((n_chunks // D) * D, n_chunks):
            iteration(ci, ci % D)

        for d in range(D):
            pltpu.make_async_copy(o_v[d], out_hbm.at[pl.ds(base, CN)], sem_o[d]).wait()

    return sc_kernel(t, idx_i, idx_j)


def kernel(x, edge_index, W, b):
    bb, c, n, _ = x.shape
    k = edge_index.shape[3]
    out_dim = W.shape[0]

    n_pad = ((n + NW * CN - 1) // (NW * CN)) * (NW * CN)

    x_t = jnp.transpose(x.reshape(c, n))
    x_t = jnp.pad(x_t, ((0, n_pad - n), (0, 0)))

    w1 = W[:, :c]
    w2 = W[:, c:]
    a_t = jnp.transpose(w1 - w2)
    b_t = jnp.transpose(w2)
    bias = b.reshape(1, out_dim)

    u, v = _tc_tables(x_t, a_t, b_t, bias)
    u = lax.bitcast_convert_type(u.reshape(n_pad, out_dim // 2, 2), jnp.int32)
    v = lax.bitcast_convert_type(v.reshape(n_pad, out_dim // 2, 2), jnp.int32)
    t = jnp.concatenate([u, v], axis=1)

    ei = edge_index.reshape(2, n * k)
    pad_e = n_pad * k - n * k
    idx_i = jnp.pad(ei[1], (0, pad_e))
    idx_j = jnp.pad(ei[0], (0, pad_e))

    out_full = _sc_aggregate(t, idx_i, idx_j, n_pad, out_dim, k)

    out_bf = lax.bitcast_convert_type(out_full[:n, :], jnp.bfloat16)
    out = jnp.transpose(out_bf.reshape(n, out_dim).astype(jnp.float32))
    return out.reshape(bb, out_dim, n, 1)


# speedup vs baseline: 1.4536x; 1.4536x over previous
import functools

import jax
import jax.numpy as jnp
from jax import lax
from jax.experimental import pallas as pl
from jax.experimental.pallas import tpu as pltpu
from jax.experimental.pallas import tpu_sc as plsc

LANES = 16
NW = 32
CN = 4


def _tc_tables(x_t, a_t, b_t, bias):
    np_, c = x_t.shape
    out = a_t.shape[1]

    def body(x_ref, a_ref, bt_ref, bias_ref, t_ref):
        xb = x_ref[...]
        t_ref[:np_, :] = (
            jnp.dot(xb, a_ref[...], preferred_element_type=jnp.float32)
            + bias_ref[...]
        )
        t_ref[np_:, :] = jnp.dot(xb, bt_ref[...], preferred_element_type=jnp.float32)

    return pl.pallas_call(
        body,
        out_shape=jax.ShapeDtypeStruct((2 * np_, out), jnp.float32),
    )(x_t, a_t, b_t, bias)


def _sc_aggregate(t, idxc, n_pad, out_dim, k):
    pw = n_pad // NW
    n_chunks = pw // CN
    ce = CN * k
    groups = out_dim // LANES
    widx = pw * k * 2

    mesh = plsc.VectorSubcoreMesh(core_axis_name="c", subcore_axis_name="s")
    D = 2

    @functools.partial(
        pl.kernel,
        mesh=mesh,
        out_type=jax.ShapeDtypeStruct((n_pad, out_dim), jnp.float32),
        scratch_types=[
            pltpu.VMEM((widx,), jnp.int32),
            [pltpu.VMEM((2 * ce, out_dim), jnp.float32)] * D,
            [pltpu.VMEM((CN, out_dim), jnp.float32)] * D,
            [pltpu.SemaphoreType.DMA] * D,
            [pltpu.SemaphoreType.DMA] * D,
        ],
    )
    def sc_kernel(t_hbm, idx_hbm, out_hbm, idx_v, g_v, o_v, sem_g, sem_o):
        wid = lax.axis_index("s") * 2 + lax.axis_index("c")
        base = wid * pw

        pltpu.sync_copy(idx_hbm.at[pl.ds(wid * widx, widx)], idx_v)

        def gather_start(ci, buf):
            pltpu.make_async_copy(
                t_hbm.at[idx_v.at[pl.ds(ci * 2 * ce, 2 * ce)]],
                g_v[buf], sem_g[buf]).start()

        def gather_wait(buf):
            pltpu.make_async_copy(
                t_hbm.at[idx_v.at[pl.ds(0, 2 * ce)]],
                g_v[buf], sem_g[buf]).wait()

        gather_start(0, 0)

        def iteration(ci, b):
            @pl.when(ci + 1 < n_chunks)
            def _():
                gather_start(ci + 1, 1 - b)

            gather_wait(b)

            @pl.when(ci >= D)
            def _():
                pltpu.make_async_copy(
                    o_v[b], out_hbm.at[pl.ds(base, CN)], sem_o[b]).wait()

            for n in range(CN):
                for g in range(groups):
                    sl = pl.ds(g * LANES, LANES)
                    s = [g_v[b][n * k + kk, sl] + g_v[b][ce + n * k + kk, sl]
                         for kk in range(k)]
                    while len(s) > 1:
                        s2 = [jnp.maximum(s[2 * i], s[2 * i + 1])
                              for i in range(len(s) // 2)]
                        s2 += s[len(s) & ~1:]
                        s = s2
                    o_v[b][n, sl] = jnp.maximum(s[0], 0.0)

            ns = base + ci * CN
            pltpu.make_async_copy(o_v[b], out_hbm.at[pl.ds(ns, CN)], sem_o[b]).start()

        def body(p, carry):
            for j in range(D):
                iteration(p * D + j, j)
            return carry

        lax.fori_loop(0, n_chunks // D, body, 0)
        for ci in range((n_chunks // D) * D, n_chunks):
            iteration(ci, ci % D)

        for d in range(D):
            pltpu.make_async_copy(o_v[d], out_hbm.at[pl.ds(base, CN)], sem_o[d]).wait()

    return sc_kernel(t, idxc)


def kernel(x, edge_index, W, b):
    bb, c, n, _ = x.shape
    k = edge_index.shape[3]
    out_dim = W.shape[0]

    n_pad = ((n + NW * CN - 1) // (NW * CN)) * (NW * CN)

    x_t = jnp.transpose(x.reshape(c, n))
    x_t = jnp.pad(x_t, ((0, n_pad - n), (0, 0)))

    w1 = W[:, :c]
    w2 = W[:, c:]
    a_t = jnp.transpose(w1 - w2)
    b_t = jnp.transpose(w2)
    bias = b.reshape(1, out_dim)

    t = _tc_tables(x_t, a_t, b_t, bias)

    ei = edge_index.reshape(2, n * k)
    pad_e = n_pad * k - n * k
    ce = CN * k
    idx_i = jnp.pad(ei[1], (0, pad_e)).reshape(-1, ce)
    idx_j = jnp.pad(ei[0], (0, pad_e)).reshape(-1, ce) + n_pad
    idxc = jnp.concatenate([idx_i, idx_j], axis=1).reshape(-1)

    out_full = _sc_aggregate(t, idxc, n_pad, out_dim, k)

    out = jnp.transpose(out_full[:n, :])
    return out.reshape(bb, out_dim, n, 1)
